# baseline (speedup 1.0000x reference)
import functools

import jax
import jax.numpy as jnp
from jax import lax
from jax.experimental import pallas as pl
from jax.experimental.pallas import tpu as pltpu
from jax.experimental.pallas import tpu_sc as plsc

DIM = 128
B = 16384
NC = 2
NS = 16
NW = NC * NS
ROWS_PW = B // NW
CH = 128
NCH = ROWS_PW // CH
NJOBS = 6 * NCH
NB = 6
PF = 3
RU = 8
NEWTON = 1

IS_ENT = (True, False, True, True, False, True)


def _lane_take(v, idx):
    dnums = lax.GatherDimensionNumbers(
        offset_dims=(), collapsed_slice_dims=(0,), start_index_map=(0,)
    )
    return lax.gather(
        v,
        idx[:, None],
        dnums,
        slice_sizes=(1,),
        mode=lax.GatherScatterMode.PROMISE_IN_BOUNDS,
    )


def _normalize_one_row(rows_v, b, r, perms):
    vs = [rows_v[b, r, pl.ds(16 * k, 16)] for k in range(DIM // 16)]
    ss = vs[0] * vs[0]
    for k in range(1, DIM // 16):
        ss = ss + vs[k] * vs[k]
    for p in perms:
        ss = ss + _lane_take(ss, p)
    i = plsc.bitcast(ss, jnp.int32)
    yi = jnp.int32(0x5F3759DF) - jnp.right_shift(i, 1)
    y = plsc.bitcast(yi, jnp.float32)
    h = 0.5 * ss
    for _ in range(NEWTON):
        y = y * (1.5 - h * y * y)
    for k in range(DIM // 16):
        rows_v[b, r, pl.ds(16 * k, 16)] = vs[k] * y


def _normalize_chunk(rows_v, b):
    lanes = lax.iota(jnp.int32, 16)
    perms = [lanes ^ (1 << bit) for bit in range(4)]

    def row_body(rg, carry):
        for u in range(RU):
            _normalize_one_row(rows_v, b, RU * rg + u, perms)
        return carry

    lax.fori_loop(0, CH // RU, row_body, 0)


def _sc_body(pos, neg, ent, rel, o0, o1, o2, o3, o4, o5, trip_v, idx_v, rows_v, *sems):
    outs = (o0, o1, o2, o3, o4, o5)
    gsem, ssem = sems[:NB], sems[NB:]
    wid = lax.axis_index("s") * NC + lax.axis_index("c")
    base0 = wid * ROWS_PW

    pltpu.sync_copy(pos.at[pl.ds(base0 * 3, ROWS_PW * 3)], trip_v.at[0])
    pltpu.sync_copy(neg.at[pl.ds(base0 * 3, ROWS_PW * 3)], trip_v.at[1])
    iota3 = lax.iota(jnp.int32, 16) * 3
    for t in range(6):
        src = jnp.full((16,), t // 3, jnp.int32)
        for g in range(ROWS_PW // 16):
            vals = plsc.load_gather(trip_v, [src, iota3 + (48 * g + t % 3)])
            c, off = divmod(16 * g, CH)
            idx_v[t * NCH + c, pl.ds(off, 16)] = vals

    ent_jobs = [(t, c) for t in (0, 2, 3, 5) for c in range(NCH)]
    rel_jobs = [(t, c) for t in (1, 4) for c in range(NCH)]
    jobs = []
    for i in range(len(rel_jobs)):
        jobs += [ent_jobs[2 * i], ent_jobs[2 * i + 1], rel_jobs[i]]

    def start_gather(j):
        t, c = jobs[j]
        table = ent if IS_ENT[t] else rel
        return pltpu.async_copy(
            table.at[idx_v.at[t * NCH + c]], rows_v.at[j % NB], gsem[j % NB]
        )

    ghandles = {}
    shandles = {}
    for j in range(PF):
        ghandles[j] = start_gather(j)
    for j in range(NJOBS):
        t, c = jobs[j]
        bb = j % NB
        ghandles[j].wait()
        if IS_ENT[t]:
            _normalize_chunk(rows_v, bb)
        shandles[j] = pltpu.async_copy(
            rows_v.at[bb], outs[t].at[pl.ds(base0 + c * CH, CH)], ssem[bb]
        )
        nj = j + PF
        if nj < NJOBS:
            if nj - NB >= 0:
                shandles[nj - NB].wait()
            ghandles[nj] = start_gather(nj)
    for j in range(NJOBS - NB, NJOBS):
        shandles[j].wait()


@jax.jit
def _sc_lookup(pos, neg, emb_ent, emb_rel):
    mesh = plsc.VectorSubcoreMesh(core_axis_name="c", subcore_axis_name="s")
    kern = functools.partial(
        pl.kernel,
        mesh=mesh,
        out_type=[jax.ShapeDtypeStruct((B, DIM), jnp.float32)] * 6,
        scratch_types=[
            pltpu.VMEM((2, ROWS_PW * 3), jnp.int32),
            pltpu.VMEM((NJOBS, CH), jnp.int32),
            pltpu.VMEM((NB, CH, DIM), jnp.float32),
        ]
        + [pltpu.SemaphoreType.DMA] * (2 * NB),
        compiler_params=pltpu.CompilerParams(needs_layout_passes=False),
    )(_sc_body)
    return kern(pos, neg, emb_ent, emb_rel)


def kernel(pos_triples, neg_triples, emb_ent, emb_rel):
    return tuple(
        _sc_lookup(
            pos_triples.astype(jnp.int32).reshape(-1),
            neg_triples.astype(jnp.int32).reshape(-1),
            emb_ent,
            emb_rel,
        )
    )


# speedup vs baseline: 1.3631x; 1.3631x over previous
import functools

import jax
import jax.numpy as jnp
from jax import lax
from jax.experimental import pallas as pl
from jax.experimental.pallas import tpu as pltpu
from jax.experimental.pallas import tpu_sc as plsc

DIM = 128
B = 16384
NC = 2
NS = 16
NW = NC * NS
ROWS_PW = B // NW
CH = 128
NCH = ROWS_PW // CH
NJOBS = 6 * NCH
NB = 7
PF = 4
RU = 8
NEWTON = 1

IS_ENT = (True, False, True, True, False, True)


def _lane_take(v, idx):
    dnums = lax.GatherDimensionNumbers(
        offset_dims=(), collapsed_slice_dims=(0,), start_index_map=(0,)
    )
    return lax.gather(
        v,
        idx[:, None],
        dnums,
        slice_sizes=(1,),
        mode=lax.GatherScatterMode.PROMISE_IN_BOUNDS,
    )


def _normalize_one_row(rows_v, b, r, perms):
    vs = [rows_v[b, r, pl.ds(16 * k, 16)] for k in range(DIM // 16)]
    ss = vs[0] * vs[0]
    for k in range(1, DIM // 16):
        ss = ss + vs[k] * vs[k]
    for p in perms:
        ss = ss + _lane_take(ss, p)
    i = plsc.bitcast(ss, jnp.int32)
    yi = jnp.int32(0x5F3759DF) - jnp.right_shift(i, 1)
    y = plsc.bitcast(yi, jnp.float32)
    h = 0.5 * ss
    for _ in range(NEWTON):
        y = y * (1.5 - h * y * y)
    for k in range(DIM // 16):
        rows_v[b, r, pl.ds(16 * k, 16)] = vs[k] * y


def _normalize_chunk(rows_v, b):
    lanes = lax.iota(jnp.int32, 16)
    perms = [lanes ^ (1 << bit) for bit in range(4)]

    def row_body(rg, carry):
        for u in range(RU):
            _normalize_one_row(rows_v, b, RU * rg + u, perms)
        return carry

    lax.fori_loop(0, CH // RU, row_body, 0)


def _sc_body(idxr, ent, rel, o0, o1, o2, o3, o4, o5, idx_v, rows_v, *sems):
    outs = (o0, o1, o2, o3, o4, o5)
    gsem, ssem = sems[:NB], sems[NB:]
    wid = lax.axis_index("s") * NC + lax.axis_index("c")
    base0 = wid * ROWS_PW

    pltpu.sync_copy(idxr.at[wid], idx_v)

    ent_jobs = [(t, c) for t in (0, 2, 3, 5) for c in range(NCH)]
    rel_jobs = [(t, c) for t in (1, 4) for c in range(NCH)]
    jobs = []
    for i in range(len(rel_jobs)):
        jobs += [ent_jobs[2 * i], ent_jobs[2 * i + 1], rel_jobs[i]]

    def start_gather(j):
        t, c = jobs[j]
        table = ent if IS_ENT[t] else rel
        return pltpu.async_copy(
            table.at[idx_v.at[t * NCH + c]], rows_v.at[j % NB], gsem[j % NB]
        )

    ghandles = {}
    shandles = {}
    for j in range(PF):
        ghandles[j] = start_gather(j)
    for j in range(NJOBS):
        t, c = jobs[j]
        bb = j % NB
        ghandles[j].wait()
        if IS_ENT[t]:
            _normalize_chunk(rows_v, bb)
        shandles[j] = pltpu.async_copy(
            rows_v.at[bb], outs[t].at[pl.ds(base0 + c * CH, CH)], ssem[bb]
        )
        nj = j + PF
        if nj < NJOBS:
            if nj - NB >= 0:
                shandles[nj - NB].wait()
            ghandles[nj] = start_gather(nj)
    for j in range(NJOBS - NB, NJOBS):
        shandles[j].wait()


@jax.jit
def _sc_lookup(idxr, emb_ent, emb_rel):
    mesh = plsc.VectorSubcoreMesh(core_axis_name="c", subcore_axis_name="s")
    kern = functools.partial(
        pl.kernel,
        mesh=mesh,
        out_type=[jax.ShapeDtypeStruct((B, DIM), jnp.float32)] * 6,
        scratch_types=[
            pltpu.VMEM((NJOBS, CH), jnp.int32),
            pltpu.VMEM((NB, CH, DIM), jnp.float32),
        ]
        + [pltpu.SemaphoreType.DMA] * (2 * NB),
        compiler_params=pltpu.CompilerParams(needs_layout_passes=False),
    )(_sc_body)
    return kern(idxr, emb_ent, emb_rel)


def kernel(pos_triples, neg_triples, emb_ent, emb_rel):
    idx6 = jnp.concatenate(
        [pos_triples.astype(jnp.int32).T, neg_triples.astype(jnp.int32).T],
        axis=0,
    )
    idxr = (
        idx6.reshape(6, NW, NCH, CH)
        .transpose(1, 0, 2, 3)
        .reshape(NW, NJOBS, CH)
    )
    return tuple(_sc_lookup(idxr, emb_ent, emb_rel))
